# baseline (speedup 1.0000x reference)
import functools

import jax
import jax.numpy as jnp
from jax import lax
from jax.experimental import pallas as pl
from jax.experimental.pallas import tpu as pltpu
from jax.experimental.pallas import tpu_sc as plsc

_B_TC = 2048
_TC_BLK = 128
_K_BLK = 128


def _tc_gather(t_tc, table):
    (BT,) = t_tc.shape
    V, D = table.shape
    grid = BT // _TC_BLK
    idx3 = t_tc.reshape(grid, 1, _TC_BLK)

    def body(idx_ref, tab_ref, o_ref):
        idx = idx_ref[0, 0, :]
        acc = jnp.zeros((_TC_BLK, D), jnp.float32)
        for k0 in range(0, V, _K_BLK):
            kw = min(_K_BLK, V - k0)
            iot = lax.broadcasted_iota(jnp.int32, (_TC_BLK, kw), 1) + k0
            oh = (iot == idx[:, None]).astype(jnp.float32)
            acc = acc + jnp.dot(
                oh,
                tab_ref[k0 : k0 + kw, :],
                preferred_element_type=jnp.float32,
                precision=lax.Precision.HIGHEST,
            )
        o_ref[...] = acc

    return pl.pallas_call(
        body,
        grid=(grid,),
        in_specs=[
            pl.BlockSpec((1, 1, _TC_BLK), lambda i: (i, 0, 0)),
            pl.BlockSpec((V, D), lambda i: (0, 0)),
        ],
        out_specs=pl.BlockSpec((_TC_BLK, D), lambda i: (i, 0)),
        out_shape=jax.ShapeDtypeStruct((BT, D), jnp.float32),
    )(idx3, table)


def kernel(t, table):
    (B,) = t.shape
    V, D = table.shape

    info = plsc.get_sparse_core_info()
    NC, NS = info.num_cores, info.num_subcores
    NW = NC * NS

    B_sc = B - _B_TC
    per_w = B_sc // NW
    n_chunks = 4
    chunk = per_w // n_chunks
    assert per_w * NW == B_sc and chunk * n_chunks == per_w
    assert chunk <= 128 and chunk % 8 == 0 and per_w % 8 == 0

    rpt = ((V + NS - 1) // NS + 7) // 8 * 8
    n_full = V // rpt
    rem = V - n_full * rpt
    assert rem % 8 == 0 and V % 8 == 0

    idx = t[:B_sc].reshape(NW, n_chunks, chunk)
    mesh = plsc.VectorSubcoreMesh(core_axis_name="c", subcore_axis_name="s")

    @functools.partial(
        pl.kernel,
        mesh=mesh,
        out_type=jax.ShapeDtypeStruct((B, D), jnp.float32),
        scratch_types=[
            pltpu.VMEM((n_chunks, chunk), jnp.int32),
            pltpu.VMEM((n_chunks, chunk, D), jnp.float32),
            pltpu.VMEM_SHARED((V, D), jnp.float32),
            pltpu.SemaphoreType.DMA,
            pltpu.SemaphoreType.DMA,
        ],
    )
    def emb(table_hbm, idx_hbm, out_hbm, idx_v, rows_v, table_sp, gsem, wsem):
        cid = lax.axis_index("c")
        sid = lax.axis_index("s")
        wid = sid * NC + cid
        idx_cp = pltpu.async_copy(idx_hbm.at[wid], idx_v, wsem)

        @pl.when(sid < n_full)
        def _():
            pltpu.sync_copy(
                table_hbm.at[pl.ds(sid * rpt, rpt)],
                table_sp.at[pl.ds(sid * rpt, rpt)],
            )

        if rem:

            @pl.when(sid == n_full)
            def _():
                pltpu.sync_copy(
                    table_hbm.at[pl.ds(n_full * rpt, rem)],
                    table_sp.at[pl.ds(n_full * rpt, rem)],
                )

        plsc.subcore_barrier()
        idx_cp.wait()
        gathers = [
            pltpu.async_copy(table_sp.at[idx_v.at[j]], rows_v.at[j], gsem)
            for j in range(n_chunks)
        ]
        writes = []
        for j in range(n_chunks):
            gathers[j].wait()
            writes.append(
                pltpu.async_copy(
                    rows_v.at[j],
                    out_hbm.at[pl.ds(wid * per_w + j * chunk, chunk)],
                    wsem,
                )
            )
        for w in writes:
            w.wait()

    out_sc = emb(table, idx)
    out_tc = _tc_gather(t[B_sc:], table)
    return lax.dynamic_update_slice(out_sc, out_tc, (B_sc, 0))


# speedup vs baseline: 1.3684x; 1.3684x over previous
import functools

import jax
import jax.numpy as jnp
from jax import lax
from jax.experimental import pallas as pl
from jax.experimental.pallas import tpu as pltpu
from jax.experimental.pallas import tpu_sc as plsc

_CHUNK = 128


def kernel(t, table):
    (B,) = t.shape
    V, D = table.shape

    info = plsc.get_sparse_core_info()
    NC, NS = info.num_cores, info.num_subcores
    NW = NC * NS

    n_chunks = B // (NW * _CHUNK)
    assert B == NW * n_chunks * _CHUNK

    rpt = ((V + NS - 1) // NS + 7) // 8 * 8
    n_full = V // rpt
    rem = V - n_full * rpt
    assert rem % 8 == 0 and V % 8 == 0

    idx = t.reshape(NW, n_chunks, _CHUNK)
    mesh = plsc.VectorSubcoreMesh(core_axis_name="c", subcore_axis_name="s")

    @functools.partial(
        pl.kernel,
        mesh=mesh,
        out_type=jax.ShapeDtypeStruct((NW, n_chunks, _CHUNK, D), jnp.float32),
        scratch_types=[
            pltpu.VMEM((n_chunks, _CHUNK), jnp.int32),
            pltpu.VMEM((n_chunks, _CHUNK, D), jnp.float32),
            pltpu.VMEM_SHARED((V, D), jnp.float32),
            pltpu.SemaphoreType.DMA,
            pltpu.SemaphoreType.DMA,
        ],
    )
    def emb(table_hbm, idx_hbm, out_hbm, idx_v, rows_v, table_sp, gsem, wsem):
        cid = lax.axis_index("c")
        sid = lax.axis_index("s")
        wid = sid * NC + cid
        idx_cp = pltpu.async_copy(idx_hbm.at[wid], idx_v, wsem)

        @pl.when(sid < n_full)
        def _():
            pltpu.sync_copy(
                table_hbm.at[pl.ds(sid * rpt, rpt)],
                table_sp.at[pl.ds(sid * rpt, rpt)],
            )

        if rem:

            @pl.when(sid == n_full)
            def _():
                pltpu.sync_copy(
                    table_hbm.at[pl.ds(n_full * rpt, rem)],
                    table_sp.at[pl.ds(n_full * rpt, rem)],
                )

        plsc.subcore_barrier()
        idx_cp.wait()
        gathers = [
            pltpu.async_copy(table_sp.at[idx_v.at[j]], rows_v.at[j], gsem)
            for j in range(n_chunks)
        ]
        writes = []
        for j in range(n_chunks):
            gathers[j].wait()
            writes.append(pltpu.async_copy(rows_v.at[j], out_hbm.at[wid, j], wsem))
        for w in writes:
            w.wait()

    return emb(table, idx).reshape(B, D)
